# baseline (speedup 1.0000x reference)
import functools

import jax
import jax.numpy as jnp
from jax import lax
from jax.experimental import pallas as pl
from jax.experimental.pallas import tpu as pltpu
from jax.experimental.pallas import tpu_sc as plsc

_NC = 2
_NS = 16
_NW = _NC * _NS
_CHUNK = 32
_NCHUNK = 16
_NBUF = 3
_BPW = _CHUNK * _NCHUNK
_D = 1024


def _sc_gather(idx_hbm, table_hbm, out_hbm, idx_v, rows_v, gsem, ssem):
    wid = lax.axis_index("s") * _NC + lax.axis_index("c")
    base = wid * _BPW

    pltpu.sync_copy(idx_hbm.at[wid], idx_v)

    scat = [None] * _NCHUNK
    gath = [None] * _NCHUNK
    for c in range(_NBUF - 1):
        gath[c] = pltpu.async_copy(table_hbm.at[idx_v.at[c]], rows_v.at[c], gsem)
    for c in range(_NCHUNK):
        buf = c % _NBUF
        nxt = c + _NBUF - 1
        if nxt < _NCHUNK:
            if nxt - _NBUF >= 0:
                scat[nxt - _NBUF].wait()
            gath[nxt] = pltpu.async_copy(
                table_hbm.at[idx_v.at[nxt]], rows_v.at[nxt % _NBUF], gsem)
        gath[c].wait()
        scat[c] = pltpu.async_copy(
            rows_v.at[buf], out_hbm.at[pl.ds(base + c * _CHUNK, _CHUNK)], ssem)
    for c in range(max(0, _NCHUNK - _NBUF), _NCHUNK):
        scat[c].wait()


@jax.jit
def _run(idx, table):
    k = functools.partial(
        pl.kernel,
        mesh=plsc.VectorSubcoreMesh(core_axis_name="c", subcore_axis_name="s"),
        out_type=jax.ShapeDtypeStruct((_NW * _BPW, _D), jnp.float32),
        scratch_types=[
            pltpu.VMEM((_NCHUNK, _CHUNK), jnp.int32),
            pltpu.VMEM((_NBUF, _CHUNK, _D), jnp.float32),
            pltpu.SemaphoreType.DMA,
            pltpu.SemaphoreType.DMA,
        ],
    )(_sc_gather)
    return k(idx, table)


def kernel(positions, table):
    b, s = positions.shape
    idx = positions.astype(jnp.int32).reshape(_NW, _NCHUNK, _CHUNK)
    out = _run(idx, table)
    return out.reshape(b, s, _D)


# speedup vs baseline: 1.0035x; 1.0035x over previous
import functools

import jax
import jax.numpy as jnp
from jax import lax
from jax.experimental import pallas as pl
from jax.experimental.pallas import tpu as pltpu
from jax.experimental.pallas import tpu_sc as plsc

_NC = 2
_NS = 16
_NW = _NC * _NS
_CHUNK = 32
_NCHUNK = 16
_NBUF = 3
_BPW = _CHUNK * _NCHUNK
_D = 1024


def _sc_gather(idx_hbm, table_hbm, out_hbm, idx_v, rows_v, gsem, ssem):
    wid = lax.axis_index("s") * _NC + lax.axis_index("c")
    base = wid * _BPW

    pltpu.sync_copy(idx_hbm.at[pl.ds(base, _BPW)], idx_v)

    scat = [None] * _NCHUNK
    gath = [None] * _NCHUNK
    for c in range(_NBUF - 1):
        gath[c] = pltpu.async_copy(
            table_hbm.at[idx_v.at[pl.ds(c * _CHUNK, _CHUNK)]], rows_v.at[c], gsem)
    for c in range(_NCHUNK):
        buf = c % _NBUF
        nxt = c + _NBUF - 1
        if nxt < _NCHUNK:
            if nxt - _NBUF >= 0:
                scat[nxt - _NBUF].wait()
            gath[nxt] = pltpu.async_copy(
                table_hbm.at[idx_v.at[pl.ds(nxt * _CHUNK, _CHUNK)]],
                rows_v.at[nxt % _NBUF], gsem)
        gath[c].wait()
        scat[c] = pltpu.async_copy(
            rows_v.at[buf], out_hbm.at[pl.ds(base + c * _CHUNK, _CHUNK)], ssem)
    for c in range(max(0, _NCHUNK - _NBUF), _NCHUNK):
        scat[c].wait()


@jax.jit
def _run(idx, table):
    k = functools.partial(
        pl.kernel,
        mesh=plsc.VectorSubcoreMesh(core_axis_name="c", subcore_axis_name="s"),
        out_type=jax.ShapeDtypeStruct((_NW * _BPW, _D), jnp.float32),
        scratch_types=[
            pltpu.VMEM((_BPW,), jnp.int32),
            pltpu.VMEM((_NBUF, _CHUNK, _D), jnp.float32),
            pltpu.SemaphoreType.DMA,
            pltpu.SemaphoreType.DMA,
        ],
    )(_sc_gather)
    return k(idx, table)


def kernel(positions, table):
    b, s = positions.shape
    idx = positions.astype(jnp.int32).reshape(-1)
    out = _run(idx, table)
    return out.reshape(b, s, _D)


# speedup vs baseline: 1.0242x; 1.0207x over previous
import functools

import jax
import jax.numpy as jnp
from jax import lax
from jax.experimental import pallas as pl
from jax.experimental.pallas import tpu as pltpu
from jax.experimental.pallas import tpu_sc as plsc

_NC = 2
_NS = 16
_NW = _NC * _NS
_CHUNK = 32
_NCHUNK = 16
_NBUF = 3
_BPW = _CHUNK * _NCHUNK
_D = 1024


def _sc_gather(idx_hbm, table_hbm, out_hbm, idx_v, rows_v, gsem, ssem):
    wid = lax.axis_index("s") * _NC + lax.axis_index("c")
    base = wid * _BPW

    pltpu.sync_copy(idx_hbm.at[pl.ds(base, _BPW)], idx_v)

    def g_start(c):
        return pltpu.async_copy(
            table_hbm.at[idx_v.at[pl.ds(c * _CHUNK, _CHUNK)]],
            rows_v.at[c % _NBUF], gsem)

    def s_start(c):
        return pltpu.async_copy(
            rows_v.at[c % _NBUF],
            out_hbm.at[pl.ds(base + c * _CHUNK, _CHUNK)], ssem)

    def g_wait():
        pltpu.make_async_copy(
            table_hbm.at[pl.ds(0, _CHUNK)], rows_v.at[0], gsem).wait()

    def s_wait():
        pltpu.make_async_copy(
            rows_v.at[0], out_hbm.at[pl.ds(base, _CHUNK)], ssem).wait()

    for c in range(_NBUF - 1):
        g_start(c)
    g_start(_NBUF - 1)
    g_wait()
    s_start(0)

    def body(c, carry):
        s_wait()
        g_start(c + _NBUF - 1)
        g_wait()
        s_start(c)
        return carry

    lax.fori_loop(1, _NCHUNK - _NBUF + 1, body, 0)

    for c in range(_NCHUNK - _NBUF + 1, _NCHUNK):
        g_wait()
        s_start(c)
    for _ in range(_NBUF):
        s_wait()


@jax.jit
def _run(idx, table):
    k = functools.partial(
        pl.kernel,
        mesh=plsc.VectorSubcoreMesh(core_axis_name="c", subcore_axis_name="s"),
        out_type=jax.ShapeDtypeStruct((_NW * _BPW, _D), jnp.float32),
        scratch_types=[
            pltpu.VMEM((_BPW,), jnp.int32),
            pltpu.VMEM((_NBUF, _CHUNK, _D), jnp.float32),
            pltpu.SemaphoreType.DMA,
            pltpu.SemaphoreType.DMA,
        ],
    )(_sc_gather)
    return k(idx, table)


def kernel(positions, table):
    b, s = positions.shape
    idx = positions.astype(jnp.int32).reshape(-1)
    out = _run(idx, table)
    return out.reshape(b, s, _D)


# speedup vs baseline: 1.0246x; 1.0004x over previous
import functools

import jax
import jax.numpy as jnp
from jax import lax
from jax.experimental import pallas as pl
from jax.experimental.pallas import tpu as pltpu
from jax.experimental.pallas import tpu_sc as plsc

_NC = 2
_NS = 16
_NW = _NC * _NS
_CHUNK = 16
_NCHUNK = 32
_NBUF = 6
_BPW = _CHUNK * _NCHUNK
_D = 1024


def _sc_gather(idx_hbm, table_hbm, out_hbm, idx_v, rows_v, gsem, ssem):
    wid = lax.axis_index("s") * _NC + lax.axis_index("c")
    base = wid * _BPW

    pltpu.sync_copy(idx_hbm.at[pl.ds(base, _BPW)], idx_v)

    def g_start(c):
        return pltpu.async_copy(
            table_hbm.at[idx_v.at[pl.ds(c * _CHUNK, _CHUNK)]],
            rows_v.at[c % _NBUF], gsem)

    def s_start(c):
        return pltpu.async_copy(
            rows_v.at[c % _NBUF],
            out_hbm.at[pl.ds(base + c * _CHUNK, _CHUNK)], ssem)

    def g_wait():
        pltpu.make_async_copy(
            table_hbm.at[pl.ds(0, _CHUNK)], rows_v.at[0], gsem).wait()

    def s_wait():
        pltpu.make_async_copy(
            rows_v.at[0], out_hbm.at[pl.ds(base, _CHUNK)], ssem).wait()

    for c in range(_NBUF - 1):
        g_start(c)
    g_start(_NBUF - 1)
    g_wait()
    s_start(0)

    def body(c, carry):
        s_wait()
        g_start(c + _NBUF - 1)
        g_wait()
        s_start(c)
        return carry

    lax.fori_loop(1, _NCHUNK - _NBUF + 1, body, 0)

    for c in range(_NCHUNK - _NBUF + 1, _NCHUNK):
        g_wait()
        s_start(c)
    for _ in range(_NBUF):
        s_wait()


@jax.jit
def _run(idx, table):
    k = functools.partial(
        pl.kernel,
        mesh=plsc.VectorSubcoreMesh(core_axis_name="c", subcore_axis_name="s"),
        out_type=jax.ShapeDtypeStruct((_NW * _BPW, _D), jnp.float32),
        scratch_types=[
            pltpu.VMEM((_BPW,), jnp.int32),
            pltpu.VMEM((_NBUF, _CHUNK, _D), jnp.float32),
            pltpu.SemaphoreType.DMA,
            pltpu.SemaphoreType.DMA,
        ],
    )(_sc_gather)
    return k(idx, table)


def kernel(positions, table):
    b, s = positions.shape
    idx = positions.astype(jnp.int32).reshape(-1)
    out = _run(idx, table)
    return out.reshape(b, s, _D)


# speedup vs baseline: 1.0269x; 1.0023x over previous
import functools

import jax
import jax.numpy as jnp
from jax import lax
from jax.experimental import pallas as pl
from jax.experimental.pallas import tpu as pltpu
from jax.experimental.pallas import tpu_sc as plsc

_NC = 2
_NS = 16
_NW = _NC * _NS
_CHUNK = 16
_NCHUNK = 32
_NBUF = 4
_BPW = _CHUNK * _NCHUNK
_D = 1024


def _sc_gather(idx_hbm, table_hbm, out_hbm, idx_v, rows_v, gsem, ssem):
    wid = lax.axis_index("s") * _NC + lax.axis_index("c")
    base = wid * _BPW

    wpr = idx_hbm.shape[1] // _BPW
    pltpu.sync_copy(
        idx_hbm.at[wid // wpr, pl.ds((wid % wpr) * _BPW, _BPW)], idx_v)

    def g_start(c):
        return pltpu.async_copy(
            table_hbm.at[idx_v.at[pl.ds(c * _CHUNK, _CHUNK)]],
            rows_v.at[c % _NBUF], gsem)

    def s_start(c):
        return pltpu.async_copy(
            rows_v.at[c % _NBUF],
            out_hbm.at[pl.ds(base + c * _CHUNK, _CHUNK)], ssem)

    def g_wait():
        pltpu.make_async_copy(
            table_hbm.at[pl.ds(0, _CHUNK)], rows_v.at[0], gsem).wait()

    def s_wait():
        pltpu.make_async_copy(
            rows_v.at[0], out_hbm.at[pl.ds(base, _CHUNK)], ssem).wait()

    for c in range(_NBUF - 1):
        g_start(c)
    g_start(_NBUF - 1)
    g_wait()
    s_start(0)

    def body(c, carry):
        s_wait()
        g_start(c + _NBUF - 1)
        g_wait()
        s_start(c)
        return carry

    lax.fori_loop(1, _NCHUNK - _NBUF + 1, body, 0)

    for c in range(_NCHUNK - _NBUF + 1, _NCHUNK):
        g_wait()
        s_start(c)
    for _ in range(_NBUF):
        s_wait()


@jax.jit
def _run(idx, table):
    k = functools.partial(
        pl.kernel,
        mesh=plsc.VectorSubcoreMesh(core_axis_name="c", subcore_axis_name="s"),
        out_type=jax.ShapeDtypeStruct((_NW * _BPW, _D), jnp.float32),
        scratch_types=[
            pltpu.VMEM((_BPW,), jnp.int32),
            pltpu.VMEM((_NBUF, _CHUNK, _D), jnp.float32),
            pltpu.SemaphoreType.DMA,
            pltpu.SemaphoreType.DMA,
        ],
    )(_sc_gather)
    return k(idx, table)


def kernel(positions, table):
    b, s = positions.shape
    out = _run(positions.astype(jnp.int32), table)
    return out.reshape(b, s, _D)


# speedup vs baseline: 1.0350x; 1.0078x over previous
import functools

import jax
import jax.numpy as jnp
from jax import lax
from jax.experimental import pallas as pl
from jax.experimental.pallas import tpu as pltpu
from jax.experimental.pallas import tpu_sc as plsc

_NC = 2
_NS = 16
_NW = _NC * _NS
_CHUNK = 32
_NCHUNK = 16
_NBUF = 2
_BPW = _CHUNK * _NCHUNK
_D = 1024


def _sc_gather(idx_hbm, table_hbm, out_hbm, idx_v, rows_v, gsem, ssem):
    wid = lax.axis_index("s") * _NC + lax.axis_index("c")
    base = wid * _BPW

    wpr = idx_hbm.shape[1] // _BPW
    pltpu.sync_copy(
        idx_hbm.at[wid // wpr, pl.ds((wid % wpr) * _BPW, _BPW)], idx_v)

    def g_start(c):
        return pltpu.async_copy(
            table_hbm.at[idx_v.at[pl.ds(c * _CHUNK, _CHUNK)]],
            rows_v.at[c % _NBUF], gsem)

    def s_start(c):
        return pltpu.async_copy(
            rows_v.at[c % _NBUF],
            out_hbm.at[pl.ds(base + c * _CHUNK, _CHUNK)], ssem)

    def g_wait():
        pltpu.make_async_copy(
            table_hbm.at[pl.ds(0, _CHUNK)], rows_v.at[0], gsem).wait()

    def s_wait():
        pltpu.make_async_copy(
            rows_v.at[0], out_hbm.at[pl.ds(base, _CHUNK)], ssem).wait()

    for c in range(_NBUF - 1):
        g_start(c)
    g_start(_NBUF - 1)
    g_wait()
    s_start(0)

    def body(c, carry):
        s_wait()
        g_start(c + _NBUF - 1)
        g_wait()
        s_start(c)
        return carry

    lax.fori_loop(1, _NCHUNK - _NBUF + 1, body, 0)

    for c in range(_NCHUNK - _NBUF + 1, _NCHUNK):
        g_wait()
        s_start(c)
    for _ in range(_NBUF):
        s_wait()


@jax.jit
def _run(idx, table):
    k = functools.partial(
        pl.kernel,
        mesh=plsc.VectorSubcoreMesh(core_axis_name="c", subcore_axis_name="s"),
        out_type=jax.ShapeDtypeStruct((_NW * _BPW, _D), jnp.float32),
        scratch_types=[
            pltpu.VMEM((_BPW,), jnp.int32),
            pltpu.VMEM((_NBUF, _CHUNK, _D), jnp.float32),
            pltpu.SemaphoreType.DMA,
            pltpu.SemaphoreType.DMA,
        ],
    )(_sc_gather)
    return k(idx, table)


def kernel(positions, table):
    b, s = positions.shape
    out = _run(positions.astype(jnp.int32), table)
    return out.reshape(b, s, _D)
